# baseline (speedup 1.0000x reference)
import functools
import jax
import jax.numpy as jnp
from jax import lax
from jax.experimental import pallas as pl
from jax.experimental.pallas import tpu as pltpu
from jax.experimental.pallas import tpu_sc as plsc

_B, _S, _D = 2, 2048, 1024
_E, _K, _F = 8, 2, 2048
_T = _B * _S
_TK = _T * _K
_M = 128
_NBLK = _TK // _M + _E
_NP = _NBLK * _M
_NBE = 80

_NTILES = 16
_PPT = _TK // _NTILES
_VPT = _PPT // 16

@functools.cache
def _sc_mesh():
    return plsc.VectorSubcoreMesh(core_axis_name="c", subcore_axis_name="s")


def _router_body(x_ref, wr_ref, br_ref, pf_ref, idx_ref):
    s = jax.lax.dot_general(x_ref[...], wr_ref[...],
                            (((1,), (1,)), ((), ())),
                            preferred_element_type=jnp.float32)
    s = s + br_ref[...]
    m = jnp.max(s, axis=-1, keepdims=True)
    ex = jnp.exp(s - m)
    p = ex / jnp.sum(ex, axis=-1, keepdims=True)
    lane = jax.lax.broadcasted_iota(jnp.int32, p.shape, 1)
    p1 = jnp.max(p, axis=-1, keepdims=True)
    i1 = jnp.min(jnp.where(p == p1, lane, _E), axis=-1, keepdims=True)
    pm = jnp.where(lane == i1, -jnp.inf, p)
    p2 = jnp.max(pm, axis=-1, keepdims=True)
    i2 = jnp.min(jnp.where(pm == p2, lane, _E), axis=-1, keepdims=True)
    pf_ref[...] = jnp.concatenate([p1, p2], axis=-1)
    idx_ref[...] = jnp.concatenate([i1, i2], axis=-1)


def _router(x_flat, Wr, br):
    return pl.pallas_call(
        _router_body,
        out_shape=(jax.ShapeDtypeStruct((_T, _K), jnp.float32),
                   jax.ShapeDtypeStruct((_T, _K), jnp.int32)),
    )(x_flat, Wr, br.reshape(1, _E))


def _layout_body(idx_hbm, pf_hbm, tok_hbm, ws_hbm, pos_hbm, blk_hbm,
                 idx_v, cnt_v, call_v, pos_v, posall_v, pfall_v,
                 tok_loc, ws_loc, blk_v, cnt_sh, pos_sh):
    core = lax.axis_index("c")
    w = lax.axis_index("s")

    @pl.when(core == 0)
    def _core0():
        lane = lax.iota(jnp.int32, 16)
        pltpu.sync_copy(idx_hbm.at[pl.ds(w * _PPT, _PPT)], idx_v)

        def count_body(i, counts):
            ev = idx_v[pl.ds(i * 16, 16)]
            for e in range(_E):
                cnt = jnp.sum(jnp.where(ev == e, 1, 0).astype(jnp.int32))
                counts = counts + jnp.where(lane == e, cnt, 0)
            return counts
        counts = lax.fori_loop(0, _VPT, count_body,
                               jnp.zeros((16,), jnp.int32))
        cnt_v[...] = counts
        pltpu.sync_copy(cnt_v, cnt_sh.at[pl.ds(w * 16, 16)])
        plsc.subcore_barrier()

        pltpu.sync_copy(cnt_sh, call_v)
        wv = jnp.full((16,), w, jnp.int32)
        total = jnp.zeros((16,), jnp.int32)
        mybase = jnp.zeros((16,), jnp.int32)
        for t in range(_NTILES):
            row = call_v[pl.ds(t * 16, 16)]
            total = total + row
            tv = jnp.full((16,), t, jnp.int32)
            mybase = mybase + jnp.where(tv < wv, row, 0)
        bpe = (total + (_M - 1)) // _M
        bcum = plsc.cumsum(bpe)
        seg = (bcum - bpe) * _M
        base0 = seg + mybase

        def pos_body(i, base):
            ev = idx_v[pl.ds(i * 16, 16)]
            posv = jnp.zeros((16,), jnp.int32)
            for e in range(_E):
                m = ev == e
                ones = jnp.where(m, 1, 0).astype(jnp.int32)
                c = plsc.cumsum(ones)
                cnt = jnp.sum(ones)
                le = lane == e
                base_e = jnp.sum(jnp.where(le, base, 0))
                posv = jnp.where(m, base_e + c - 1, posv)
                base = base + jnp.where(le, cnt, 0)
            pos_v[pl.ds(i * 16, 16)] = posv
            return base
        lax.fori_loop(0, _VPT, pos_body, base0)
        pltpu.sync_copy(pos_v, pos_sh.at[pl.ds(w * _PPT, _PPT)])
        plsc.subcore_barrier()

        @pl.when(w == 0)
        def _tile0():
            pltpu.sync_copy(pos_sh, posall_v)
            pltpu.sync_copy(pf_hbm, pfall_v)
            zi = jnp.zeros((16,), jnp.int32)
            zf = jnp.zeros((16,), jnp.float32)

            def zero_body(j, _):
                tok_loc[pl.ds(j * 16, 16)] = zi
                ws_loc[pl.ds(j * 16, 16)] = zf
                return 0
            lax.fori_loop(0, _NP // 16, zero_body, 0)

            def scat_body(i, _):
                pv = posall_v[pl.ds(i * 16, 16)]
                gpi = jnp.full((16,), i * 16, jnp.int32) + lane
                tk = lax.shift_right_logical(gpi, 1)
                wsv = pfall_v[pl.ds(i * 16, 16)]
                plsc.store_scatter(tok_loc, [pv], tk)
                plsc.store_scatter(ws_loc, [pv], wsv)
                return 0
            lax.fori_loop(0, _TK // 16, scat_body, 0)

            for bv in range(_NBE // 16):
                bids = jnp.full((16,), bv * 16, jnp.int32) + lane
                acc = jnp.zeros((16,), jnp.int32)
                for e in range(_E):
                    bcum_e = jnp.sum(jnp.where(lane == e, bcum, 0))
                    acc = acc + jnp.where(bids >= bcum_e, 1, 0)
                blk_v[pl.ds(bv * 16, 16)] = jnp.minimum(acc, _E - 1)

            pltpu.sync_copy(tok_loc, tok_hbm)
            pltpu.sync_copy(ws_loc, ws_hbm)
            pltpu.sync_copy(posall_v, pos_hbm)
            pltpu.sync_copy(blk_v, blk_hbm)


def _layout(idx_flat, pf_flat):
    f = pl.kernel(
        _layout_body,
        mesh=_sc_mesh(),
        compiler_params=pltpu.CompilerParams(needs_layout_passes=False),
        out_type=(jax.ShapeDtypeStruct((_NP,), jnp.int32),
                  jax.ShapeDtypeStruct((_NP,), jnp.float32),
                  jax.ShapeDtypeStruct((_TK,), jnp.int32),
                  jax.ShapeDtypeStruct((_NBE,), jnp.int32)),
        scratch_types=[
            pltpu.VMEM((_PPT,), jnp.int32),
            pltpu.VMEM((16,), jnp.int32),
            pltpu.VMEM((16 * _NTILES,), jnp.int32),
            pltpu.VMEM((_PPT,), jnp.int32),
            pltpu.VMEM((_TK,), jnp.int32),
            pltpu.VMEM((_TK,), jnp.float32),
            pltpu.VMEM((_NP,), jnp.int32),
            pltpu.VMEM((_NP,), jnp.float32),
            pltpu.VMEM((_NBE,), jnp.int32),
            pltpu.VMEM_SHARED((16 * _NTILES,), jnp.int32),
            pltpu.VMEM_SHARED((_TK,), jnp.int32),
        ],
    )
    return f(idx_flat, pf_flat)


_GCH = 48
_NPH = _NP // 2
_RPW = _NPH // 32
_NCH = _RPW // _GCH


def _dispatch_body(x_hbm, tok_hbm, xs_hbm, idx_v, ra_v, rb_v, gsem, wsem):
    wid = lax.axis_index("s") * 2 + lax.axis_index("c")
    base = wid * _RPW
    pltpu.sync_copy(tok_hbm.at[pl.ds(base, _RPW)], idx_v)
    bufs = (ra_v, rb_v)
    g = [None] * _NCH
    wr = [None] * _NCH
    g[0] = pltpu.async_copy(x_hbm.at[idx_v.at[pl.ds(0, _GCH)]], ra_v, gsem)
    for c in range(_NCH):
        cur = bufs[c % 2]
        nxt = bufs[(c + 1) % 2]
        g[c].wait()
        if c >= 1:
            wr[c - 1].wait()
        if c + 1 < _NCH:
            g[c + 1] = pltpu.async_copy(
                x_hbm.at[idx_v.at[pl.ds((c + 1) * _GCH, _GCH)]], nxt, gsem)
        wr[c] = pltpu.async_copy(
            cur, xs_hbm.at[pl.ds(base + c * _GCH, _GCH)], wsem)
    wr[_NCH - 1].wait()


def _dispatch(x_flat, tok_pad):
    f = pl.kernel(
        _dispatch_body,
        mesh=_sc_mesh(),
        compiler_params=pltpu.CompilerParams(needs_layout_passes=False,
                                             use_tc_tiling_on_sc=True),
        out_type=jax.ShapeDtypeStruct((_NPH, _D), jnp.float32),
        scratch_types=[
            pltpu.VMEM((_RPW,), jnp.int32),
            pltpu.VMEM((_GCH, _D), jnp.float32),
            pltpu.VMEM((_GCH, _D), jnp.float32),
            pltpu.SemaphoreType.DMA,
            pltpu.SemaphoreType.DMA,
        ],
    )
    return f(x_flat, tok_pad)


def _layer1_body(be_ref, xs_ref, w1_ref, b1_ref, h_ref):
    del be_ref
    h = jax.lax.dot_general(xs_ref[...], w1_ref[0], (((1,), (1,)), ((), ())),
                            preferred_element_type=jnp.float32)
    h_ref[...] = h + b1_ref[0, 0]


def _layer1(block_expert_half, xs_half, W1, b1):
    grid_spec = pltpu.PrefetchScalarGridSpec(
        num_scalar_prefetch=1,
        grid=(_NBLK // 2,),
        in_specs=[
            pl.BlockSpec((_M, _D), lambda b, be: (b, 0)),
            pl.BlockSpec((1, _F, _D), lambda b, be: (be[b], 0, 0)),
            pl.BlockSpec((1, 1, _F), lambda b, be: (be[b], 0, 0)),
        ],
        out_specs=pl.BlockSpec((_M, _F), lambda b, be: (b, 0)),
    )
    return pl.pallas_call(
        _layer1_body,
        grid_spec=grid_spec,
        out_shape=jax.ShapeDtypeStruct((_NPH, _F), jnp.float32),
    )(block_expert_half, xs_half, W1, b1)


def _layer2_body(be_ref, ha_ref, hb_ref, ws_ref, wg_ref, bg_ref, w2_ref,
                 b2_ref, out_ref):
    del be_ref
    b = pl.program_id(0)
    h = jnp.where(b < _NBLK // 2, ha_ref[...], hb_ref[...])
    g = jax.lax.dot_general(h, wg_ref[0], (((1,), (1,)), ((), ())),
                            preferred_element_type=jnp.float32)
    g = jnp.maximum(g + bg_ref[0, 0], 0.0)
    y = jax.lax.dot_general(g, w2_ref[0], (((1,), (1,)), ((), ())),
                            preferred_element_type=jnp.float32)
    y = y + b2_ref[0, 0]
    out_ref[...] = y * ws_ref[...]


def _layer2(block_expert, h1a, h1b, ws, Wg, bg, W2, b2):
    nh = _NBLK // 2
    grid_spec = pltpu.PrefetchScalarGridSpec(
        num_scalar_prefetch=1,
        grid=(_NBLK,),
        in_specs=[
            pl.BlockSpec((_M, _F),
                         lambda b, be: (jnp.minimum(b, nh - 1), 0)),
            pl.BlockSpec((_M, _F),
                         lambda b, be: (jnp.maximum(b - nh, 0), 0)),
            pl.BlockSpec((_M, 1), lambda b, be: (b, 0)),
            pl.BlockSpec((1, _F, _F), lambda b, be: (be[b], 0, 0)),
            pl.BlockSpec((1, 1, _F), lambda b, be: (be[b], 0, 0)),
            pl.BlockSpec((1, _D, _F), lambda b, be: (be[b], 0, 0)),
            pl.BlockSpec((1, 1, _D), lambda b, be: (be[b], 0, 0)),
        ],
        out_specs=pl.BlockSpec((_M, _D), lambda b, be: (b, 0)),
    )
    return pl.pallas_call(
        _layer2_body,
        grid_spec=grid_spec,
        compiler_params=pltpu.CompilerParams(
            vmem_limit_bytes=64 * 1024 * 1024),
        out_shape=jax.ShapeDtypeStruct((_NP, _D), jnp.float32),
    )(block_expert, h1a, h1b, ws, Wg, bg, W2, b2)


_CCH = 16
_TPW = _T // 32
_NCC = _TPW // _CCH


def _combine_body(ys_hbm, pos_hbm, out_hbm, idx_v, aa_v, ab_v, oa_v, ob_v,
                  gsem, wsem):
    wid = lax.axis_index("s") * 2 + lax.axis_index("c")
    tbase = wid * _TPW
    pltpu.sync_copy(pos_hbm.at[pl.ds(tbase * _K, _TPW * _K)], idx_v)
    abufs = (aa_v, ab_v)
    obufs = (oa_v, ob_v)
    g = [None] * _NCC
    wr = [None] * _NCC
    g[0] = pltpu.async_copy(
        ys_hbm.at[idx_v.at[pl.ds(0, _CCH * _K)]], aa_v, gsem)
    for c in range(_NCC):
        cur = abufs[c % 2]
        ocur = obufs[c % 2]
        g[c].wait()
        if c + 1 < _NCC:
            g[c + 1] = pltpu.async_copy(
                ys_hbm.at[idx_v.at[pl.ds((c + 1) * _CCH * _K, _CCH * _K)]],
                abufs[(c + 1) % 2], gsem)
        if c >= 2:
            wr[c - 2].wait()

        def add_body(j, _):
            for v in range(_D // 16):
                sl = pl.ds(v * 16, 16)
                ocur[j, sl] = cur[2 * j, sl] + cur[2 * j + 1, sl]
            return 0
        lax.fori_loop(0, _CCH, add_body, 0)
        wr[c] = pltpu.async_copy(
            ocur, out_hbm.at[pl.ds(tbase + c * _CCH, _CCH)], wsem)
    wr[_NCC - 2].wait()
    wr[_NCC - 1].wait()


def _combine(ys, pos):
    f = pl.kernel(
        _combine_body,
        mesh=_sc_mesh(),
        compiler_params=pltpu.CompilerParams(needs_layout_passes=False,
                                             use_tc_tiling_on_sc=True),
        out_type=jax.ShapeDtypeStruct((_T, _D), jnp.float32),
        scratch_types=[
            pltpu.VMEM((_TPW * _K,), jnp.int32),
            pltpu.VMEM((_CCH * _K, _D), jnp.float32),
            pltpu.VMEM((_CCH * _K, _D), jnp.float32),
            pltpu.VMEM((_CCH, _D), jnp.float32),
            pltpu.VMEM((_CCH, _D), jnp.float32),
            pltpu.SemaphoreType.DMA,
            pltpu.SemaphoreType.DMA,
        ],
    )
    return f(ys, pos)


def kernel(x, Wr, br, W1, b1, Wg, bg, W2, b2):
    x_flat = x.reshape(_T, _D)
    pf, idxf = _router(x_flat, Wr, br)
    tok_pad, ws_pad, pos, blk = _layout(idxf.reshape(_TK), pf.reshape(_TK))
    nh = _NBLK // 2
    xs_a = _dispatch(x_flat, tok_pad[:_NPH])
    xs_b = _dispatch(x_flat, tok_pad[_NPH:])
    b1r = b1.reshape(_E, 1, _F)
    h1a = _layer1(blk[:nh], xs_a, W1, b1r)
    h1b = _layer1(blk[nh:_NBLK], xs_b, W1, b1r)
    ys = _layer2(blk[:_NBLK], h1a, h1b, ws_pad.reshape(_NP, 1),
                 Wg, bg.reshape(_E, 1, _F), W2, b2.reshape(_E, 1, _D))
    out = _combine(ys, pos)
    return out.reshape(_B, _S, _D)


# speedup vs baseline: 1.3123x; 1.3123x over previous
import functools
import jax
import jax.numpy as jnp
from jax import lax
from jax.experimental import pallas as pl
from jax.experimental.pallas import tpu as pltpu
from jax.experimental.pallas import tpu_sc as plsc

_B, _S, _D = 2, 2048, 1024
_E, _K, _F = 8, 2, 2048
_T = _B * _S
_TK = _T * _K
_M = 256
_NBLK = _TK // _M + _E
_NP = _NBLK * _M
_NBE = 48

_NTILES = 16
_PPT = _TK // _NTILES
_VPT = _PPT // 16

@functools.cache
def _sc_mesh():
    return plsc.VectorSubcoreMesh(core_axis_name="c", subcore_axis_name="s")


def _router_body(x_ref, wr_ref, br_ref, pf_ref, idx_ref):
    s = jax.lax.dot_general(x_ref[...], wr_ref[...],
                            (((1,), (1,)), ((), ())),
                            preferred_element_type=jnp.float32)
    s = s + br_ref[...]
    m = jnp.max(s, axis=-1, keepdims=True)
    ex = jnp.exp(s - m)
    p = ex / jnp.sum(ex, axis=-1, keepdims=True)
    lane = jax.lax.broadcasted_iota(jnp.int32, p.shape, 1)
    p1 = jnp.max(p, axis=-1, keepdims=True)
    i1 = jnp.min(jnp.where(p == p1, lane, _E), axis=-1, keepdims=True)
    pm = jnp.where(lane == i1, -jnp.inf, p)
    p2 = jnp.max(pm, axis=-1, keepdims=True)
    i2 = jnp.min(jnp.where(pm == p2, lane, _E), axis=-1, keepdims=True)
    pf_ref[...] = jnp.concatenate([p1, p2], axis=-1)
    idx_ref[...] = jnp.concatenate([i1, i2], axis=-1)


def _router(x_flat, Wr, br):
    return pl.pallas_call(
        _router_body,
        out_shape=(jax.ShapeDtypeStruct((_T, _K), jnp.float32),
                   jax.ShapeDtypeStruct((_T, _K), jnp.int32)),
    )(x_flat, Wr, br.reshape(1, _E))


def _layout_body(idx_hbm, pf_hbm, tok_hbm, ws_hbm, pos_hbm, blk_hbm,
                 idx_v, cnt_v, call_v, pos_v, posall_v, pfall_v,
                 tok_loc, ws_loc, blk_v, cnt_sh, pos_sh):
    core = lax.axis_index("c")
    w = lax.axis_index("s")

    @pl.when(core == 0)
    def _core0():
        lane = lax.iota(jnp.int32, 16)
        pltpu.sync_copy(idx_hbm.at[pl.ds(w * _PPT, _PPT)], idx_v)

        def count_body(i, counts):
            ev = idx_v[pl.ds(i * 16, 16)]
            for e in range(_E):
                cnt = jnp.sum(jnp.where(ev == e, 1, 0).astype(jnp.int32))
                counts = counts + jnp.where(lane == e, cnt, 0)
            return counts
        counts = lax.fori_loop(0, _VPT, count_body,
                               jnp.zeros((16,), jnp.int32))
        cnt_v[...] = counts
        pltpu.sync_copy(cnt_v, cnt_sh.at[pl.ds(w * 16, 16)])
        plsc.subcore_barrier()

        pltpu.sync_copy(cnt_sh, call_v)
        wv = jnp.full((16,), w, jnp.int32)
        total = jnp.zeros((16,), jnp.int32)
        mybase = jnp.zeros((16,), jnp.int32)
        for t in range(_NTILES):
            row = call_v[pl.ds(t * 16, 16)]
            total = total + row
            tv = jnp.full((16,), t, jnp.int32)
            mybase = mybase + jnp.where(tv < wv, row, 0)
        bpe = (total + (_M - 1)) // _M
        bcum = plsc.cumsum(bpe)
        seg = (bcum - bpe) * _M
        base0 = seg + mybase

        def pos_body(i, base):
            ev = idx_v[pl.ds(i * 16, 16)]
            posv = jnp.zeros((16,), jnp.int32)
            for e in range(_E):
                m = ev == e
                ones = jnp.where(m, 1, 0).astype(jnp.int32)
                c = plsc.cumsum(ones)
                cnt = jnp.sum(ones)
                le = lane == e
                base_e = jnp.sum(jnp.where(le, base, 0))
                posv = jnp.where(m, base_e + c - 1, posv)
                base = base + jnp.where(le, cnt, 0)
            pos_v[pl.ds(i * 16, 16)] = posv
            return base
        lax.fori_loop(0, _VPT, pos_body, base0)
        pltpu.sync_copy(pos_v, pos_sh.at[pl.ds(w * _PPT, _PPT)])
        plsc.subcore_barrier()

        @pl.when(w == 0)
        def _tile0():
            pltpu.sync_copy(pos_sh, posall_v)
            pltpu.sync_copy(pf_hbm, pfall_v)
            zi = jnp.zeros((16,), jnp.int32)
            zf = jnp.zeros((16,), jnp.float32)

            def zero_body(j, _):
                tok_loc[pl.ds(j * 16, 16)] = zi
                ws_loc[pl.ds(j * 16, 16)] = zf
                return 0
            lax.fori_loop(0, _NP // 16, zero_body, 0)

            def scat_body(i, _):
                pv = posall_v[pl.ds(i * 16, 16)]
                gpi = jnp.full((16,), i * 16, jnp.int32) + lane
                tk = lax.shift_right_logical(gpi, 1)
                wsv = pfall_v[pl.ds(i * 16, 16)]
                plsc.store_scatter(tok_loc, [pv], tk)
                plsc.store_scatter(ws_loc, [pv], wsv)
                return 0
            lax.fori_loop(0, _TK // 16, scat_body, 0)

            for bv in range(_NBE // 16):
                bids = jnp.full((16,), bv * 16, jnp.int32) + lane
                acc = jnp.zeros((16,), jnp.int32)
                for e in range(_E):
                    bcum_e = jnp.sum(jnp.where(lane == e, bcum, 0))
                    acc = acc + jnp.where(bids >= bcum_e, 1, 0)
                blk_v[pl.ds(bv * 16, 16)] = jnp.minimum(acc, _E - 1)

            pltpu.sync_copy(tok_loc, tok_hbm)
            pltpu.sync_copy(ws_loc, ws_hbm)
            pltpu.sync_copy(posall_v, pos_hbm)
            pltpu.sync_copy(blk_v, blk_hbm)


def _layout(idx_flat, pf_flat):
    f = pl.kernel(
        _layout_body,
        mesh=_sc_mesh(),
        compiler_params=pltpu.CompilerParams(needs_layout_passes=False),
        out_type=(jax.ShapeDtypeStruct((_NP,), jnp.int32),
                  jax.ShapeDtypeStruct((_NP,), jnp.float32),
                  jax.ShapeDtypeStruct((_TK,), jnp.int32),
                  jax.ShapeDtypeStruct((_NBE,), jnp.int32)),
        scratch_types=[
            pltpu.VMEM((_PPT,), jnp.int32),
            pltpu.VMEM((16,), jnp.int32),
            pltpu.VMEM((16 * _NTILES,), jnp.int32),
            pltpu.VMEM((_PPT,), jnp.int32),
            pltpu.VMEM((_TK,), jnp.int32),
            pltpu.VMEM((_TK,), jnp.float32),
            pltpu.VMEM((_NP,), jnp.int32),
            pltpu.VMEM((_NP,), jnp.float32),
            pltpu.VMEM((_NBE,), jnp.int32),
            pltpu.VMEM_SHARED((16 * _NTILES,), jnp.int32),
            pltpu.VMEM_SHARED((_TK,), jnp.int32),
        ],
    )
    return f(idx_flat, pf_flat)


_GCH = 40
_NPH = _NP // 2
_RPW = _NPH // 32
_NCH = _RPW // _GCH


def _dispatch_body(x_hbm, tok_hbm, xs_hbm, idx_v, ra_v, rb_v, gsem, wsem):
    wid = lax.axis_index("s") * 2 + lax.axis_index("c")
    base = wid * _RPW
    pltpu.sync_copy(tok_hbm.at[pl.ds(base, _RPW)], idx_v)
    bufs = (ra_v, rb_v)
    g = [None] * _NCH
    wr = [None] * _NCH
    g[0] = pltpu.async_copy(x_hbm.at[idx_v.at[pl.ds(0, _GCH)]], ra_v, gsem)
    for c in range(_NCH):
        cur = bufs[c % 2]
        nxt = bufs[(c + 1) % 2]
        g[c].wait()
        if c >= 1:
            wr[c - 1].wait()
        if c + 1 < _NCH:
            g[c + 1] = pltpu.async_copy(
                x_hbm.at[idx_v.at[pl.ds((c + 1) * _GCH, _GCH)]], nxt, gsem)
        wr[c] = pltpu.async_copy(
            cur, xs_hbm.at[pl.ds(base + c * _GCH, _GCH)], wsem)
    wr[_NCH - 1].wait()


def _dispatch(x_flat, tok_pad):
    f = pl.kernel(
        _dispatch_body,
        mesh=_sc_mesh(),
        compiler_params=pltpu.CompilerParams(needs_layout_passes=False,
                                             use_tc_tiling_on_sc=True),
        out_type=jax.ShapeDtypeStruct((_NPH, _D), jnp.float32),
        scratch_types=[
            pltpu.VMEM((_RPW,), jnp.int32),
            pltpu.VMEM((_GCH, _D), jnp.float32),
            pltpu.VMEM((_GCH, _D), jnp.float32),
            pltpu.SemaphoreType.DMA,
            pltpu.SemaphoreType.DMA,
        ],
    )
    return f(x_flat, tok_pad)


def _layer1_body(be_ref, xs_ref, w1_ref, b1_ref, h_ref):
    del be_ref
    h = jax.lax.dot_general(xs_ref[...], w1_ref[0], (((1,), (1,)), ((), ())),
                            preferred_element_type=jnp.float32)
    h_ref[...] = h + b1_ref[0, 0]


def _layer1(block_expert_half, xs_half, W1, b1):
    grid_spec = pltpu.PrefetchScalarGridSpec(
        num_scalar_prefetch=1,
        grid=(_NBLK // 2,),
        in_specs=[
            pl.BlockSpec((_M, _D), lambda b, be: (b, 0)),
            pl.BlockSpec((1, _F, _D), lambda b, be: (be[b], 0, 0)),
            pl.BlockSpec((1, 1, _F), lambda b, be: (be[b], 0, 0)),
        ],
        out_specs=pl.BlockSpec((_M, _F), lambda b, be: (b, 0)),
    )
    return pl.pallas_call(
        _layer1_body,
        grid_spec=grid_spec,
        out_shape=jax.ShapeDtypeStruct((_NPH, _F), jnp.float32),
    )(block_expert_half, xs_half, W1, b1)


def _layer2_body(be_ref, ha_ref, hb_ref, ws_ref, wg_ref, bg_ref, w2_ref,
                 b2_ref, out_ref):
    del be_ref
    b = pl.program_id(0)
    h = jnp.where(b < _NBLK // 2, ha_ref[...], hb_ref[...])
    g = jax.lax.dot_general(h, wg_ref[0], (((1,), (1,)), ((), ())),
                            preferred_element_type=jnp.float32)
    g = jnp.maximum(g + bg_ref[0, 0], 0.0)
    y = jax.lax.dot_general(g, w2_ref[0], (((1,), (1,)), ((), ())),
                            preferred_element_type=jnp.float32)
    y = y + b2_ref[0, 0]
    out_ref[...] = y * ws_ref[...]


def _layer2(block_expert, h1a, h1b, ws, Wg, bg, W2, b2):
    nh = _NBLK // 2
    grid_spec = pltpu.PrefetchScalarGridSpec(
        num_scalar_prefetch=1,
        grid=(_NBLK,),
        in_specs=[
            pl.BlockSpec((_M, _F),
                         lambda b, be: (jnp.minimum(b, nh - 1), 0)),
            pl.BlockSpec((_M, _F),
                         lambda b, be: (jnp.maximum(b - nh, 0), 0)),
            pl.BlockSpec((_M, 1), lambda b, be: (b, 0)),
            pl.BlockSpec((1, _F, _F), lambda b, be: (be[b], 0, 0)),
            pl.BlockSpec((1, 1, _F), lambda b, be: (be[b], 0, 0)),
            pl.BlockSpec((1, _D, _F), lambda b, be: (be[b], 0, 0)),
            pl.BlockSpec((1, 1, _D), lambda b, be: (be[b], 0, 0)),
        ],
        out_specs=pl.BlockSpec((_M, _D), lambda b, be: (b, 0)),
    )
    return pl.pallas_call(
        _layer2_body,
        grid_spec=grid_spec,
        compiler_params=pltpu.CompilerParams(
            vmem_limit_bytes=64 * 1024 * 1024),
        out_shape=jax.ShapeDtypeStruct((_NP, _D), jnp.float32),
    )(block_expert, h1a, h1b, ws, Wg, bg, W2, b2)


_CCH = 16
_TPW = _T // 32
_NCC = _TPW // _CCH


def _combine_body(ys_hbm, pos_hbm, out_hbm, idx_v, aa_v, ab_v, oa_v, ob_v,
                  gsem, wsem):
    wid = lax.axis_index("s") * 2 + lax.axis_index("c")
    tbase = wid * _TPW
    pltpu.sync_copy(pos_hbm.at[pl.ds(tbase * _K, _TPW * _K)], idx_v)
    abufs = (aa_v, ab_v)
    obufs = (oa_v, ob_v)
    g = [None] * _NCC
    wr = [None] * _NCC
    g[0] = pltpu.async_copy(
        ys_hbm.at[idx_v.at[pl.ds(0, _CCH * _K)]], aa_v, gsem)
    for c in range(_NCC):
        cur = abufs[c % 2]
        ocur = obufs[c % 2]
        g[c].wait()
        if c + 1 < _NCC:
            g[c + 1] = pltpu.async_copy(
                ys_hbm.at[idx_v.at[pl.ds((c + 1) * _CCH * _K, _CCH * _K)]],
                abufs[(c + 1) % 2], gsem)
        if c >= 2:
            wr[c - 2].wait()

        def add_body(j, _):
            for v in range(_D // 16):
                sl = pl.ds(v * 16, 16)
                ocur[j, sl] = cur[2 * j, sl] + cur[2 * j + 1, sl]
            return 0
        lax.fori_loop(0, _CCH, add_body, 0)
        wr[c] = pltpu.async_copy(
            ocur, out_hbm.at[pl.ds(tbase + c * _CCH, _CCH)], wsem)
    wr[_NCC - 2].wait()
    wr[_NCC - 1].wait()


def _combine(ys, pos):
    f = pl.kernel(
        _combine_body,
        mesh=_sc_mesh(),
        compiler_params=pltpu.CompilerParams(needs_layout_passes=False,
                                             use_tc_tiling_on_sc=True),
        out_type=jax.ShapeDtypeStruct((_T, _D), jnp.float32),
        scratch_types=[
            pltpu.VMEM((_TPW * _K,), jnp.int32),
            pltpu.VMEM((_CCH * _K, _D), jnp.float32),
            pltpu.VMEM((_CCH * _K, _D), jnp.float32),
            pltpu.VMEM((_CCH, _D), jnp.float32),
            pltpu.VMEM((_CCH, _D), jnp.float32),
            pltpu.SemaphoreType.DMA,
            pltpu.SemaphoreType.DMA,
        ],
    )
    return f(ys, pos)


def kernel(x, Wr, br, W1, b1, Wg, bg, W2, b2):
    x_flat = x.reshape(_T, _D)
    pf, idxf = _router(x_flat, Wr, br)
    tok_pad, ws_pad, pos, blk = _layout(idxf.reshape(_TK), pf.reshape(_TK))
    nh = _NBLK // 2
    xs_a = _dispatch(x_flat, tok_pad[:_NPH])
    xs_b = _dispatch(x_flat, tok_pad[_NPH:])
    b1r = b1.reshape(_E, 1, _F)
    h1a = _layer1(blk[:nh], xs_a, W1, b1r)
    h1b = _layer1(blk[nh:_NBLK], xs_b, W1, b1r)
    ys = _layer2(blk[:_NBLK], h1a, h1b, ws_pad.reshape(_NP, 1),
                 Wg, bg.reshape(_E, 1, _F), W2, b2.reshape(_E, 1, _D))
    out = _combine(ys, pos)
    return out.reshape(_B, _S, _D)
